# baseline (speedup 1.0000x reference)
import functools

import jax
import jax.numpy as jnp
from jax import lax
from jax.experimental import pallas as pl
from jax.experimental.pallas import tpu as pltpu
from jax.experimental.pallas import tpu_sc as plsc

_B = 256
_N_PV = 1400
_F = 64
_EMB = 32
_OUTC = 2 * _F + _F + _EMB + 2
_NROWS = _B * _N_PV

_NC = 2
_NS = 16
_NW = _NC * _NS
_RPW = _NROWS // _NW
_R = 320
_NCH = _RPW // _R

_PBLK = 8
_GRID = _N_PV // _PBLK


def _sc_gather(idx, table128):
    mesh = plsc.VectorSubcoreMesh(core_axis_name="c", subcore_axis_name="s")

    @functools.partial(
        pl.kernel,
        mesh=mesh,
        compiler_params=pltpu.CompilerParams(use_tc_tiling_on_sc=False),
        out_type=jax.ShapeDtypeStruct((_NROWS, 128), jnp.float32),
        scratch_types=[
            pltpu.VMEM((2, _R), jnp.int32),
            pltpu.VMEM((2, _R, 128), jnp.float32),
            pltpu.SemaphoreType.DMA((2,)),
            pltpu.SemaphoreType.DMA((2,)),
            pltpu.SemaphoreType.DMA((2,)),
        ],
    )
    def k(idx_hbm, table_hbm, out_hbm, idx_v, emb_v,
          sem_idx, sem_g, sem_out):
        wid = lax.axis_index("s") * _NC + lax.axis_index("c")
        base = wid * _RPW

        def start_idx(c, p):
            return pltpu.async_copy(idx_hbm.at[pl.ds(base + c * _R, _R)],
                                    idx_v.at[p], sem_idx.at[p])

        ins = {0: start_idx(0, 0)}
        outs = {}
        for c in range(_NCH):
            p = c & 1
            q = 1 - p
            if c + 1 < _NCH:
                ins[q] = start_idx(c + 1, q)
            ins.pop(p).wait()
            if c >= 2:
                outs.pop(p).wait()
            pltpu.async_copy(table_hbm.at[idx_v.at[p]], emb_v.at[p],
                             sem_g.at[p]).wait()
            outs[p] = pltpu.async_copy(
                emb_v.at[p], out_hbm.at[pl.ds(base + c * _R, _R)],
                sem_out.at[p])
        for h in outs.values():
            h.wait()

    return k(idx, table128)


def _fix(v):
    return jnp.where(v != v, jnp.float32(0.0), v)


def _tc_body(y_ref, x_ref, t_ref, az_ref, el_ref, out_ref):
    t = _fix(t_ref[...])
    az = _fix(az_ref[...])
    el = _fix(el_ref[...])
    zeros = jnp.zeros((_EMB, _B), jnp.float32)
    for pp in range(_PBLK):
        y = _fix(y_ref[pp])
        x = _fix(x_ref[pp])
        out_ref[:, pp, :] = jnp.concatenate(
            [y, x, t, zeros, az, el], axis=0)


def _tc_assemble(y_t, x_t, t_t, az_r, el_r):
    return pl.pallas_call(
        _tc_body,
        grid=(_GRID,),
        in_specs=[
            pl.BlockSpec((_PBLK, _F, _B), lambda i: (i, 0, 0)),
            pl.BlockSpec((_PBLK, _F, _B), lambda i: (i, 0, 0)),
            pl.BlockSpec((_F, _B), lambda i: (0, 0)),
            pl.BlockSpec((1, _B), lambda i: (0, 0)),
            pl.BlockSpec((1, _B), lambda i: (0, 0)),
        ],
        out_specs=pl.BlockSpec((_OUTC, _PBLK, _B), lambda i: (0, i, 0)),
        out_shape=jax.ShapeDtypeStruct((_OUTC, _N_PV, _B), jnp.float32),
    )(y_t, x_t, t_t, az_r, el_r)


def _tc_embed_body(emb_ref, outin_ref, out_ref):
    del outin_ref
    for pp in range(_PBLK):
        e_rows = _fix(emb_ref[:, pp, pl.ds(0, _EMB)])
        out_ref[:, pp, :] = jnp.transpose(e_rows, (1, 0))


def _tc_embed(emb3, out1):
    return pl.pallas_call(
        _tc_embed_body,
        grid=(_GRID,),
        in_specs=[
            pl.BlockSpec((_B, _PBLK, 128), lambda i: (0, i, 0)),
            pl.BlockSpec(memory_space=pltpu.MemorySpace.HBM),
        ],
        out_specs=pl.BlockSpec((_EMB, _PBLK, _B), lambda i: (3 * _F // _EMB, i, 0)),
        out_shape=jax.ShapeDtypeStruct((_OUTC, _N_PV, _B), jnp.float32),
        input_output_aliases={1: 0},
    )(emb3, out1)


def kernel(pv_y_osgb_fourier, pv_x_osgb_fourier, pv_system_row_number,
           pv_time_utc_fourier, pv_x_osgb, solar_azimuth, solar_elevation,
           pv_embedding):
    idx = pv_system_row_number.reshape(-1).astype(jnp.int32)
    table128 = jnp.pad(pv_embedding, ((0, 0), (0, 128 - _EMB)))
    emb = _sc_gather(idx, table128)
    emb3 = emb.reshape(_B, _N_PV, 128)
    y_t = jnp.transpose(pv_y_osgb_fourier, (1, 2, 0))
    x_t = jnp.transpose(pv_x_osgb_fourier, (1, 2, 0))
    t_t = jnp.transpose(pv_time_utc_fourier[:, 12], (1, 0))
    az_r = solar_azimuth[:, 12].reshape(1, _B)
    el_r = solar_elevation[:, 12].reshape(1, _B)
    out1 = _tc_assemble(y_t, x_t, t_t, az_r, el_r)
    out_t = _tc_embed(emb3, out1)
    return jnp.transpose(out_t, (2, 1, 0))


# speedup vs baseline: 1.1078x; 1.1078x over previous
import functools

import jax
import jax.numpy as jnp
from jax import lax
from jax.experimental import pallas as pl
from jax.experimental.pallas import tpu as pltpu
from jax.experimental.pallas import tpu_sc as plsc

_B = 256
_N_PV = 1400
_F = 64
_EMB = 32
_OUTC = 2 * _F + _F + _EMB + 2
_NROWS = _B * _N_PV

_NC = 2
_NS = 16
_NW = _NC * _NS
_RPW = _NROWS // _NW
_R = 320
_NCH = _RPW // _R

_PBLK = 8
_GRID = _N_PV // _PBLK


def _sc_gather(idx, table128):
    mesh = plsc.VectorSubcoreMesh(core_axis_name="c", subcore_axis_name="s")

    @functools.partial(
        pl.kernel,
        mesh=mesh,
        compiler_params=pltpu.CompilerParams(use_tc_tiling_on_sc=False),
        out_type=jax.ShapeDtypeStruct((_NROWS, 128), jnp.float32),
        scratch_types=[
            pltpu.VMEM((2, _R), jnp.int32),
            pltpu.VMEM((2, _R, 128), jnp.float32),
            pltpu.SemaphoreType.DMA((2,)),
            pltpu.SemaphoreType.DMA((2,)),
            pltpu.SemaphoreType.DMA((2,)),
        ],
    )
    def k(idx_hbm, table_hbm, out_hbm, idx_v, emb_v,
          sem_idx, sem_g, sem_out):
        wid = lax.axis_index("s") * _NC + lax.axis_index("c")
        base = wid * _RPW

        def start_idx(c, p):
            return pltpu.async_copy(idx_hbm.at[pl.ds(base + c * _R, _R)],
                                    idx_v.at[p], sem_idx.at[p])

        ins = {0: start_idx(0, 0)}
        outs = {}
        for c in range(_NCH):
            p = c & 1
            q = 1 - p
            if c + 1 < _NCH:
                ins[q] = start_idx(c + 1, q)
            ins.pop(p).wait()
            if c >= 2:
                outs.pop(p).wait()
            pltpu.async_copy(table_hbm.at[idx_v.at[p]], emb_v.at[p],
                             sem_g.at[p]).wait()
            outs[p] = pltpu.async_copy(
                emb_v.at[p], out_hbm.at[pl.ds(base + c * _R, _R)],
                sem_out.at[p])
        for h in outs.values():
            h.wait()

    return k(idx, table128)


def _fix(v):
    return jnp.where(v != v, jnp.float32(0.0), v)


def _tc_body(y_ref, x_ref, emb_ref, t_ref, az_ref, el_ref, out_ref):
    t = _fix(t_ref[...])
    az = _fix(az_ref[...])
    el = _fix(el_ref[...])
    for pp in range(_PBLK):
        y = _fix(y_ref[pp])
        x = _fix(x_ref[pp])
        e_rows = _fix(emb_ref[:, pp, pl.ds(0, _EMB)])
        e = jnp.transpose(e_rows, (1, 0))
        out_ref[:, pp, :] = jnp.concatenate(
            [y, x, t, e, az, el], axis=0)


def _tc_assemble(y_t, x_t, emb3, t_t, az_r, el_r):
    return pl.pallas_call(
        _tc_body,
        grid=(_GRID,),
        in_specs=[
            pl.BlockSpec((_PBLK, _F, _B), lambda i: (i, 0, 0)),
            pl.BlockSpec((_PBLK, _F, _B), lambda i: (i, 0, 0)),
            pl.BlockSpec((_B, _PBLK, 128), lambda i: (0, i, 0)),
            pl.BlockSpec((_F, _B), lambda i: (0, 0)),
            pl.BlockSpec((1, _B), lambda i: (0, 0)),
            pl.BlockSpec((1, _B), lambda i: (0, 0)),
        ],
        out_specs=pl.BlockSpec((_OUTC, _PBLK, _B), lambda i: (0, i, 0)),
        out_shape=jax.ShapeDtypeStruct((_OUTC, _N_PV, _B), jnp.float32),
    )(y_t, x_t, emb3, t_t, az_r, el_r)


def kernel(pv_y_osgb_fourier, pv_x_osgb_fourier, pv_system_row_number,
           pv_time_utc_fourier, pv_x_osgb, solar_azimuth, solar_elevation,
           pv_embedding):
    idx = pv_system_row_number.reshape(-1).astype(jnp.int32)
    table128 = jnp.pad(pv_embedding, ((0, 0), (0, 128 - _EMB)))
    emb = _sc_gather(idx, table128)
    emb3 = emb.reshape(_B, _N_PV, 128)
    y_t = jnp.transpose(pv_y_osgb_fourier, (1, 2, 0))
    x_t = jnp.transpose(pv_x_osgb_fourier, (1, 2, 0))
    t_t = jnp.transpose(pv_time_utc_fourier[:, 12], (1, 0))
    az_r = solar_azimuth[:, 12].reshape(1, _B)
    el_r = solar_elevation[:, 12].reshape(1, _B)
    out_t = _tc_assemble(y_t, x_t, emb3, t_t, az_r, el_r)
    return jnp.transpose(out_t, (2, 1, 0))
